# baseline (speedup 1.0000x reference)
import jax
import jax.numpy as jnp
import numpy as np
from jax.experimental import pallas as pl
from jax.experimental.pallas import tpu as pltpu

FDIM = 128
NSH = 9
MAXL = 2
_BLOCK = 2000


def _interaction_kernel(inv_ref, evt_ref, w1_ref, w2_ref, r_ref, b_ref,
                        out_inv_ref, out_evt_ref):
    att_inv = inv_ref[...] * 2.0
    att_evt = evt_ref[0] * 2.0
    sqt = att_evt * att_evt
    ev_invt = jnp.dot(r_ref[...], sqt, preferred_element_type=jnp.float32)
    ev_inv = ev_invt.T
    t_all = (jnp.dot(att_inv, w1_ref[...], preferred_element_type=jnp.float32)
             + jnp.dot(ev_inv, w2_ref[...], preferred_element_type=jnp.float32)
             + b_ref[...])
    out_inv_ref[...] = att_inv + t_all[:, :FDIM]
    rept = t_all[:, FDIM:FDIM + NSH].T
    out_evt_ref[0] = att_evt + rept * att_evt


def kernel(inv_features, ev_features, senders, receivers, sh_vectors, lengths, cutoffs,
           Wi_r1, bi_r1, Wi_r2, bi_r2, Wi_e1, bi_e1, Wi_e2, bi_e2,
           We_r1, be_r1, We_r2, be_r2, We_e1, be_e1, We_e2, be_e2,
           W_int, b_int):
    n = inv_features.shape[0]
    r = np.zeros((MAXL + 1, NSH), np.float32)
    r[0, 0] = 1.0
    r[1, 1:4] = 1.0
    r[2, 4:9] = 1.0
    r = jnp.asarray(r)

    wt = W_int.T
    w1 = jnp.concatenate([wt[:FDIM, :FDIM], wt[:FDIM, FDIM:] @ r], axis=1)
    w2 = jnp.concatenate([wt[FDIM:, :FDIM], wt[FDIM:, FDIM:] @ r], axis=1)
    bias = jnp.concatenate([b_int[:FDIM], b_int[FDIM:] @ r])[None, :]

    nblk = n // _BLOCK
    evt = ev_features.reshape(nblk, _BLOCK, NSH).transpose(0, 2, 1)

    new_inv, new_evt = pl.pallas_call(
        _interaction_kernel,
        grid=(nblk,),
        compiler_params=pltpu.CompilerParams(
            dimension_semantics=[pltpu.GridDimensionSemantics.PARALLEL]),
        in_specs=[
            pl.BlockSpec((_BLOCK, FDIM), lambda i: (i, 0)),
            pl.BlockSpec((1, NSH, _BLOCK), lambda i: (i, 0, 0)),
            pl.BlockSpec(w1.shape, lambda i: (0, 0)),
            pl.BlockSpec(w2.shape, lambda i: (0, 0)),
            pl.BlockSpec(r.shape, lambda i: (0, 0)),
            pl.BlockSpec(bias.shape, lambda i: (0, 0)),
        ],
        out_specs=[
            pl.BlockSpec((_BLOCK, FDIM), lambda i: (i, 0)),
            pl.BlockSpec((1, NSH, _BLOCK), lambda i: (i, 0, 0)),
        ],
        out_shape=[
            jax.ShapeDtypeStruct((n, FDIM), jnp.float32),
            jax.ShapeDtypeStruct((nblk, NSH, _BLOCK), jnp.float32),
        ],
    )(inv_features, evt, w1, w2, r, bias)
    return (new_inv, new_evt.transpose(0, 2, 1).reshape(n, NSH))


# speedup vs baseline: 1.3913x; 1.3913x over previous
import jax
import jax.numpy as jnp
import numpy as np
from jax.experimental import pallas as pl

FDIM = 128
NSH = 9
MAXL = 2
_BLOCK = 10000


def _interaction_kernel(inv_ref, evt_ref, w1_ref, w2_ref, r_ref, b_ref,
                        out_inv_ref, out_evt_ref):
    inv = inv_ref[...]
    evt = evt_ref[0]
    sqt = evt * evt
    ev_invt = jnp.dot(r_ref[...], sqt, preferred_element_type=jnp.float32)
    ev_inv = ev_invt.T
    t_all = (jnp.dot(inv, w1_ref[...], preferred_element_type=jnp.float32)
             + jnp.dot(ev_inv, w2_ref[...], preferred_element_type=jnp.float32)
             + b_ref[...])
    out_inv_ref[...] = inv * 2.0 + t_all[:, :FDIM]
    out_evt_ref[0] = evt * t_all[:, FDIM:FDIM + NSH].T


def kernel(inv_features, ev_features, senders, receivers, sh_vectors, lengths, cutoffs,
           Wi_r1, bi_r1, Wi_r2, bi_r2, Wi_e1, bi_e1, Wi_e2, bi_e2,
           We_r1, be_r1, We_r2, be_r2, We_e1, be_e1, We_e2, be_e2,
           W_int, b_int):
    n = inv_features.shape[0]
    r = np.zeros((MAXL + 1, NSH), np.float32)
    r[0, 0] = 1.0
    r[1, 1:4] = 1.0
    r[2, 4:9] = 1.0
    r = jnp.asarray(r)

    wt = W_int.T
    w1 = jnp.concatenate([2.0 * wt[:FDIM, :FDIM],
                          4.0 * (wt[:FDIM, FDIM:] @ r)], axis=1)
    w2 = jnp.concatenate([wt[FDIM:, :FDIM],
                          2.0 * (wt[FDIM:, FDIM:] @ r)], axis=1)
    bias = jnp.concatenate([b_int[:FDIM],
                            2.0 * (b_int[FDIM:] @ r) + 2.0])[None, :]
    r = 4.0 * r

    nblk = n // _BLOCK
    evt = ev_features.reshape(nblk, _BLOCK, NSH).transpose(0, 2, 1)

    new_inv, new_evt = pl.pallas_call(
        _interaction_kernel,
        grid=(nblk,),
        in_specs=[
            pl.BlockSpec((_BLOCK, FDIM), lambda i: (i, 0)),
            pl.BlockSpec((1, NSH, _BLOCK), lambda i: (i, 0, 0)),
            pl.BlockSpec(w1.shape, lambda i: (0, 0)),
            pl.BlockSpec(w2.shape, lambda i: (0, 0)),
            pl.BlockSpec(r.shape, lambda i: (0, 0)),
            pl.BlockSpec(bias.shape, lambda i: (0, 0)),
        ],
        out_specs=[
            pl.BlockSpec((_BLOCK, FDIM), lambda i: (i, 0)),
            pl.BlockSpec((1, NSH, _BLOCK), lambda i: (i, 0, 0)),
        ],
        out_shape=[
            jax.ShapeDtypeStruct((n, FDIM), jnp.float32),
            jax.ShapeDtypeStruct((nblk, NSH, _BLOCK), jnp.float32),
        ],
    )(inv_features, evt, w1, w2, r, bias)
    return (new_inv, new_evt.transpose(0, 2, 1).reshape(n, NSH))
